# Initial kernel scaffold; baseline (speedup 1.0000x reference)
import jax
import jax.numpy as jnp
from jax.experimental import pallas as pl


def kernel(contexts, targets, ctx_table, tgt_table):
    raise NotImplementedError("write your pallas kernel here")


# speedup vs baseline: 1.8551x; 1.8551x over previous
import functools

import jax
import jax.numpy as jnp
from jax import lax
from jax.experimental import pallas as pl
from jax.experimental.pallas import tpu as pltpu
from jax.experimental.pallas import tpu_sc as plsc

B = 16384
C = 20
D = 64
NW = 32
BPW = B // NW
NB = 32
NCHUNK = BPW // NB
RPC = NB * C
IDXW = 80
IDXR = RPC // IDXW
NG = RPC // 16


def _rsqrt(nsq):
    i = plsc.bitcast(nsq, jnp.int32)
    y = plsc.bitcast(jnp.int32(0x5F3759DF) - (i >> 1), jnp.float32)
    for _ in range(3):
        y = y * (1.5 - 0.5 * nsq * y * y)
    return y


def _scales(rows, idx_v, scale_v, inv):
    for g in range(NG):
        r0 = g * 16
        rowi = r0 + lax.iota(jnp.int32, 16)

        def body(j, nsq):
            col = jnp.full((16,), j, jnp.int32)
            v = plsc.load_gather(rows, [rowi, col])
            return nsq + v * v

        nsq = lax.fori_loop(0, D, body, jnp.zeros((16,), jnp.float32))
        s = jnp.where(nsq > 1.0, _rsqrt(nsq), 1.0)
        iv = idx_v[g // 5, pl.ds((g % 5) * 16, 16)]
        s = jnp.where(iv == 0, 0.0, s) * inv
        scale_v[pl.ds(r0, 16)] = s


def _sc_body(ctx_i, tgt_i, ctx_table, tgt_table, out,
             idx_c, idx_t, rows_c, rows_t, scale_c, scale_t, ce, simb, sem):
    wid = lax.axis_index("s") * 2 + lax.axis_index("c")

    def chunk(ch, carry):
        base = wid * BPW + ch * NB
        irow = pl.multiple_of(base * C // IDXW, 8)
        pltpu.sync_copy(ctx_i.at[pl.ds(irow, IDXR)], idx_c)
        pltpu.sync_copy(tgt_i.at[pl.ds(irow, IDXR)], idx_t)

        copies = []
        for j in range(IDXR):
            copies.append(pltpu.async_copy(
                ctx_table.at[idx_c.at[j]], rows_c.at[pl.ds(j * IDXW, IDXW)], sem))
            copies.append(pltpu.async_copy(
                tgt_table.at[idx_t.at[j]], rows_t.at[pl.ds(j * IDXW, IDXW)], sem))
        for cp in copies:
            cp.wait()

        _scales(rows_c, idx_c, scale_c, 1.0 / C)
        _scales(rows_t, idx_t, scale_t, 1.0)

        for b in range(NB):
            def cbody(c, accs):
                r = b * C + c
                s = plsc.load_gather(scale_c, [jnp.full((16,), r, jnp.int32)])
                return tuple(accs[k] + s * rows_c[r, pl.ds(k * 16, 16)]
                             for k in range(4))

            accs = lax.fori_loop(0, C, cbody,
                                 tuple(jnp.zeros((16,), jnp.float32)
                                       for _ in range(4)))
            for k in range(4):
                ce[b, pl.ds(k * 16, 16)] = accs[k]

        for g in range(NG):
            r0 = g * 16
            rowi = r0 + lax.iota(jnp.int32, 16)
            bidx = rowi // C

            def dbody(d, acc):
                col = jnp.full((16,), d, jnp.int32)
                tv = plsc.load_gather(rows_t, [rowi, col])
                cv = plsc.load_gather(ce, [bidx, col])
                return acc + tv * cv

            acc = lax.fori_loop(0, D, dbody, jnp.zeros((16,), jnp.float32))
            simb[pl.ds(r0, 16)] = acc * scale_t[pl.ds(r0, 16)]

        pltpu.sync_copy(simb, out.at[pl.ds(pl.multiple_of(base * C, 8), RPC)])
        return carry

    lax.fori_loop(0, NCHUNK, chunk, 0)


@jax.jit
def _run(ctx_i, tgt_i, ctx_table, tgt_table):
    mesh = plsc.VectorSubcoreMesh(core_axis_name="c", subcore_axis_name="s")
    f = pl.kernel(
        _sc_body,
        mesh=mesh,
        compiler_params=pltpu.CompilerParams(use_tc_tiling_on_sc=False,
                                             needs_layout_passes=False),
        out_type=jax.ShapeDtypeStruct((B * C,), jnp.float32),
        scratch_types=[
            pltpu.VMEM((IDXR, IDXW), jnp.int32),
            pltpu.VMEM((IDXR, IDXW), jnp.int32),
            pltpu.VMEM((RPC, D), jnp.float32),
            pltpu.VMEM((RPC, D), jnp.float32),
            pltpu.VMEM((RPC,), jnp.float32),
            pltpu.VMEM((RPC,), jnp.float32),
            pltpu.VMEM((NB, D), jnp.float32),
            pltpu.VMEM((RPC,), jnp.float32),
            pltpu.SemaphoreType.DMA,
        ],
    )
    return f(ctx_i, tgt_i, ctx_table, tgt_table)


def kernel(contexts, targets, ctx_table, tgt_table):
    ctx_i = contexts.reshape(-1, IDXW)
    tgt_i = targets.reshape(-1, IDXW)
    out = _run(ctx_i, tgt_i, ctx_table, tgt_table)
    return out.reshape(B, C)


# speedup vs baseline: 5.2101x; 2.8085x over previous
import jax
import jax.numpy as jnp
from jax import lax
from jax.experimental import pallas as pl
from jax.experimental.pallas import tpu as pltpu
from jax.experimental.pallas import tpu_sc as plsc

B = 16384
C = 20
D = 64
NW = 32
BPW = B // NW
NB = 32
NCHUNK = BPW // NB
RPC = NB * C
IDXW = 80
IDXR = RPC // IDXW
NG = RPC // 16


def _rsqrt(nsq):
    i = plsc.bitcast(nsq, jnp.int32)
    y = plsc.bitcast(jnp.int32(0x5F3759DF) - (i >> 1), jnp.float32)
    for _ in range(3):
        y = y * (1.5 - 0.5 * nsq * y * y)
    return y


def _scales(rows, idx_ref, scale_ref, inv, ptmp):
    def grp(g, carry):
        r0 = g * 16
        for i in range(16):
            r = r0 + i
            v0 = rows[r, pl.ds(0, 16)]
            v1 = rows[r, pl.ds(16, 16)]
            v2 = rows[r, pl.ds(32, 16)]
            v3 = rows[r, pl.ds(48, 16)]
            ptmp[i, pl.ds(0, 16)] = v0 * v0 + v1 * v1 + v2 * v2 + v3 * v3
        lanei = lax.iota(jnp.int32, 16)
        nsq = jnp.zeros((16,), jnp.float32)
        for l in range(16):
            nsq = nsq + plsc.load_gather(ptmp, [lanei, jnp.full((16,), l, jnp.int32)])
        s = jnp.where(nsq > 1.0, _rsqrt(nsq), 1.0)
        rflat = r0 + lanei
        iv = plsc.load_gather(idx_ref, [rflat // IDXW, rflat % IDXW])
        s = jnp.where(iv == 0, 0.0, s) * inv
        scale_ref[pl.ds(r0, 16)] = s
        return carry

    lax.fori_loop(0, NG, grp, 0)


def _sc_body(ctx_i, tgt_i, ctx_table, tgt_table, out,
             idx_c, idx_t, rows_c, rows_t, scale_c, scale_t, ce, pbuf, ptmp,
             simb, sem):
    wid = lax.axis_index("s") * 2 + lax.axis_index("c")

    def chunk(ch, carry):
        base = wid * BPW + ch * NB
        irow = pl.multiple_of(base * C // IDXW, 8)
        pltpu.sync_copy(ctx_i.at[pl.ds(irow, IDXR)], idx_c)
        pltpu.sync_copy(tgt_i.at[pl.ds(irow, IDXR)], idx_t)

        copies = []
        for j in range(IDXR):
            copies.append(pltpu.async_copy(
                ctx_table.at[idx_c.at[j]], rows_c.at[pl.ds(j * IDXW, IDXW)], sem))
            copies.append(pltpu.async_copy(
                tgt_table.at[idx_t.at[j]], rows_t.at[pl.ds(j * IDXW, IDXW)], sem))
        for cp in copies:
            cp.wait()

        _scales(rows_c, idx_c, scale_c, 1.0 / C, ptmp)
        _scales(rows_t, idx_t, scale_t, 1.0, ptmp)

        def cbody(b, carry):
            accs = [jnp.zeros((16,), jnp.float32) for _ in range(4)]
            for c in range(C):
                r = b * C + c
                s = plsc.load_gather(scale_c, [jnp.full((16,), r, jnp.int32)])
                for k in range(4):
                    accs[k] = accs[k] + s * rows_c[r, pl.ds(k * 16, 16)]
            for k in range(4):
                ce[b, pl.ds(k * 16, 16)] = accs[k]
            return carry

        lax.fori_loop(0, NB, cbody, 0)

        def dbody(b, carry):
            cv = [ce[b, pl.ds(k * 16, 16)] for k in range(4)]
            for t in range(C):
                r = b * C + t
                p = cv[0] * rows_t[r, pl.ds(0, 16)]
                for k in range(1, 4):
                    p = p + cv[k] * rows_t[r, pl.ds(k * 16, 16)]
                pbuf[r, pl.ds(0, 16)] = p
            return carry

        lax.fori_loop(0, NB, dbody, 0)

        def fbody(g, carry):
            r0 = g * 16
            lanei = lax.iota(jnp.int32, 16)
            acc = jnp.zeros((16,), jnp.float32)
            for l in range(16):
                acc = acc + plsc.load_gather(
                    pbuf, [r0 + lanei, jnp.full((16,), l, jnp.int32)])
            simb[pl.ds(r0, 16)] = acc * scale_t[pl.ds(r0, 16)]
            return carry

        lax.fori_loop(0, NG, fbody, 0)

        pltpu.sync_copy(simb, out.at[pl.ds(pl.multiple_of(base * C, 8), RPC)])
        return carry

    lax.fori_loop(0, NCHUNK, chunk, 0)


@jax.jit
def _run(ctx_i, tgt_i, ctx_table, tgt_table):
    mesh = plsc.VectorSubcoreMesh(core_axis_name="c", subcore_axis_name="s")
    f = pl.kernel(
        _sc_body,
        mesh=mesh,
        compiler_params=pltpu.CompilerParams(use_tc_tiling_on_sc=False,
                                             needs_layout_passes=False),
        out_type=jax.ShapeDtypeStruct((B * C,), jnp.float32),
        scratch_types=[
            pltpu.VMEM((IDXR, IDXW), jnp.int32),
            pltpu.VMEM((IDXR, IDXW), jnp.int32),
            pltpu.VMEM((RPC, D), jnp.float32),
            pltpu.VMEM((RPC, D), jnp.float32),
            pltpu.VMEM((RPC,), jnp.float32),
            pltpu.VMEM((RPC,), jnp.float32),
            pltpu.VMEM((NB, D), jnp.float32),
            pltpu.VMEM((RPC, 16), jnp.float32),
            pltpu.VMEM((16, 16), jnp.float32),
            pltpu.VMEM((RPC,), jnp.float32),
            pltpu.SemaphoreType.DMA,
        ],
    )
    return f(ctx_i, tgt_i, ctx_table, tgt_table)


def kernel(contexts, targets, ctx_table, tgt_table):
    ctx_i = contexts.reshape(-1, IDXW)
    tgt_i = targets.reshape(-1, IDXW)
    out = _run(ctx_i, tgt_i, ctx_table, tgt_table)
    return out.reshape(B, C)
